# baseline (speedup 1.0000x reference)
import functools

import jax
import jax.numpy as jnp
from jax import lax
from jax.experimental import pallas as pl
from jax.experimental.pallas import tpu as pltpu
from jax.experimental.pallas import tpu_sc as plsc

_R = 64
_V = 32000
_L = 16
_NC = 2
_NS = 16
_NW = _NC * _NS
_ROWS_PER_W = _R // _NW


_G_CACHE = None


def _gumbel():
    global _G_CACHE
    if _G_CACHE is None:
        with jax.ensure_compile_time_eval():
            key = jax.random.key(42)
            u = jax.random.uniform(key, (_R, _V), dtype=jnp.float32,
                                   minval=1e-20, maxval=1.0)
            _G_CACHE = jnp.ravel(-jnp.log(-jnp.log(u)))
    return _G_CACHE


@functools.partial(
    pl.kernel,
    out_type=jax.ShapeDtypeStruct((_R, _L), jnp.int32),
    mesh=plsc.VectorSubcoreMesh(core_axis_name="c", subcore_axis_name="s",
                                num_cores=_NC, num_subcores=_NS),
    compiler_params=pltpu.CompilerParams(needs_layout_passes=False),
    scratch_types=[
        pltpu.VMEM((_V // 5,), jnp.float32),
        pltpu.VMEM((_V // 5,), jnp.float32),
        pltpu.VMEM((_V // 5,), jnp.float32),
        pltpu.VMEM((_V // 5,), jnp.float32),
        pltpu.VMEM((_V // 5,), jnp.float32),
        pltpu.VMEM((_V // 5,), jnp.float32),
        pltpu.VMEM((_V // 5,), jnp.float32),
        pltpu.VMEM((_V // 5,), jnp.float32),
        pltpu.VMEM((_V // 5,), jnp.float32),
        pltpu.VMEM((_V // 5,), jnp.float32),
        pltpu.VMEM((_V // 5,), jnp.float32),
        pltpu.VMEM((_V // 5,), jnp.float32),
        pltpu.VMEM((_L,), jnp.int32),
        pltpu.SemaphoreType.DMA,
        pltpu.SemaphoreType.DMA,
        pltpu.SemaphoreType.DMA,
        pltpu.SemaphoreType.DMA,
        pltpu.SemaphoreType.DMA,
        pltpu.SemaphoreType.DMA,
        pltpu.SemaphoreType.DMA,
        pltpu.SemaphoreType.DMA,
        pltpu.SemaphoreType.DMA,
        pltpu.SemaphoreType.DMA,
        pltpu.SemaphoreType.DMA,
        pltpu.SemaphoreType.DMA,
    ],
)
def _argmax_rows(x_hbm, g_hbm, out_hbm, xb0, xb1, xb2, xb3, xb4, xb5,
                 gb0, gb1, gb2, gb3, gb4, gb5, res_v,
                 sx0, sx1, sx2, sx3, sx4, sx5, sg0, sg1, sg2, sg3, sg4, sg5):
    cid = lax.axis_index("c")
    sid = lax.axis_index("s")
    wid = sid * _NC + cid
    quart = _V // 5
    nch = quart // _L
    sx = (sx0, sx1, sx2, sx3, sx4, sx5)
    sg = (sg0, sg1, sg2, sg3, sg4, sg5)
    xbufs = (xb0, xb1, xb2, xb3, xb4, xb5)
    gbufs = (gb0, gb1, gb2, gb3, gb4, gb5)
    ntask = 5 * _ROWS_PER_W

    def start(t):
        row = wid * _ROWS_PER_W + t // 5
        q = t % 5
        b = t % 6
        src = pl.ds(q * quart, quart)
        gsrc = pl.ds(row * _V + q * quart, quart)
        return (pltpu.async_copy(x_hbm.at[row, src], xbufs[b], sx[b]),
                pltpu.async_copy(g_hbm.at[gsrc], gbufs[b], sg[b]))

    init = (jnp.full((_L,), -jnp.inf, jnp.float32),
            jnp.zeros((_L,), jnp.int32))
    inflight = [start(0), start(1), start(2), start(3), start(4)]
    carry = init
    for t in range(ntask):
        cps = inflight.pop(0)
        cps[0].wait()
        cps[1].wait()
        if t + 5 < ntask:
            inflight.append(start(t + 5))
        q = t % 5
        b = t % 6
        xv = xbufs[b]
        gv = gbufs[b]
        base = q * quart

        @plsc.parallel_loop(0, nch, unroll=8, carry=carry)
        def carry(i, c, xv=xv, gv=gv, base=base):
            bv, bi = c
            off = i * _L
            v = xv[pl.ds(off, _L)] + gv[pl.ds(off, _L)]
            idx = (base + off) + lax.iota(jnp.int32, _L)
            m = v > bv
            return jnp.where(m, v, bv), jnp.where(m, idx, bi)
        if q == 4:
            bv, bi = carry
            nk, bv1 = lax.sort_key_val(-bi, bv)
            _, bi2 = lax.sort_key_val(bv1, -nk)
            res_v[...] = bi2
            pltpu.sync_copy(res_v, out_hbm.at[wid * _ROWS_PER_W + t // 5])
            carry = init


def kernel(outputs, number_actions=1):
    del number_actions
    out = _argmax_rows(outputs, _gumbel())
    return out[:, 15:16].astype(jnp.int64)


# speedup vs baseline: 1.0139x; 1.0139x over previous
import functools

import jax
import jax.numpy as jnp
from jax import lax
from jax.experimental import pallas as pl
from jax.experimental.pallas import tpu as pltpu
from jax.experimental.pallas import tpu_sc as plsc

_R = 64
_V = 32000
_L = 16
_NC = 2
_NS = 16
_NW = _NC * _NS
_ROWS_PER_W = _R // _NW


_G_CACHE = None


def _gumbel():
    global _G_CACHE
    if _G_CACHE is None:
        with jax.ensure_compile_time_eval():
            key = jax.random.key(42)
            u = jax.random.uniform(key, (_R, _V), dtype=jnp.float32,
                                   minval=1e-20, maxval=1.0)
            _G_CACHE = jnp.ravel(-jnp.log(-jnp.log(u)))
    return _G_CACHE


@functools.partial(
    pl.kernel,
    out_type=jax.ShapeDtypeStruct((_R, _L), jnp.int32),
    mesh=plsc.VectorSubcoreMesh(core_axis_name="c", subcore_axis_name="s",
                                num_cores=_NC, num_subcores=_NS),
    compiler_params=pltpu.CompilerParams(needs_layout_passes=False),
    scratch_types=[
        pltpu.VMEM((_V // 5,), jnp.float32),
        pltpu.VMEM((_V // 5,), jnp.float32),
        pltpu.VMEM((_V // 5,), jnp.float32),
        pltpu.VMEM((_V // 5,), jnp.float32),
        pltpu.VMEM((_V // 5,), jnp.float32),
        pltpu.VMEM((_V // 5,), jnp.float32),
        pltpu.VMEM((_V // 5,), jnp.float32),
        pltpu.VMEM((_V // 5,), jnp.float32),
        pltpu.VMEM((_L,), jnp.int32),
        pltpu.SemaphoreType.DMA,
        pltpu.SemaphoreType.DMA,
        pltpu.SemaphoreType.DMA,
        pltpu.SemaphoreType.DMA,
        pltpu.SemaphoreType.DMA,
        pltpu.SemaphoreType.DMA,
        pltpu.SemaphoreType.DMA,
        pltpu.SemaphoreType.DMA,
    ],
)
def _argmax_rows(x_hbm, g_hbm, out_hbm, xb0, xb1, xb2, xb3,
                 gb0, gb1, gb2, gb3, res_v,
                 sx0, sx1, sx2, sx3, sg0, sg1, sg2, sg3):
    cid = lax.axis_index("c")
    sid = lax.axis_index("s")
    wid = sid * _NC + cid
    quart = _V // 5
    nch = quart // _L
    sx = (sx0, sx1, sx2, sx3)
    sg = (sg0, sg1, sg2, sg3)
    xbufs = (xb0, xb1, xb2, xb3)
    gbufs = (gb0, gb1, gb2, gb3)
    ntask = 5 * _ROWS_PER_W

    def start(t):
        row = wid * _ROWS_PER_W + t // 5
        q = t % 5
        b = t % 4
        src = pl.ds(q * quart, quart)
        gsrc = pl.ds(row * _V + q * quart, quart)
        return (pltpu.async_copy(x_hbm.at[row, src], xbufs[b], sx[b]),
                pltpu.async_copy(g_hbm.at[gsrc], gbufs[b], sg[b]))

    init = (jnp.full((_L,), -jnp.inf, jnp.float32),
            jnp.zeros((_L,), jnp.int32))
    inflight = [start(0), start(1), start(2)]
    carry = init
    for t in range(ntask):
        cps = inflight.pop(0)
        cps[0].wait()
        cps[1].wait()
        if t + 3 < ntask:
            inflight.append(start(t + 3))
        q = t % 5
        b = t % 4
        xv = xbufs[b]
        gv = gbufs[b]
        base = q * quart

        @plsc.parallel_loop(0, nch, unroll=8, carry=carry)
        def carry(i, c, xv=xv, gv=gv, base=base):
            bv, bi = c
            off = i * _L
            v = xv[pl.ds(off, _L)] + gv[pl.ds(off, _L)]
            idx = (base + off) + lax.iota(jnp.int32, _L)
            m = v > bv
            return jnp.where(m, v, bv), jnp.where(m, idx, bi)
        if q == 4:
            bv, bi = carry
            nk, bv1 = lax.sort_key_val(-bi, bv)
            _, bi2 = lax.sort_key_val(bv1, -nk)
            res_v[...] = bi2
            pltpu.sync_copy(res_v, out_hbm.at[wid * _ROWS_PER_W + t // 5])
            carry = init


def kernel(outputs, number_actions=1):
    del number_actions
    out = _argmax_rows(outputs, _gumbel())
    return out[:, 15:16].astype(jnp.int64)
